# baseline (speedup 1.0000x reference)
import functools

import jax
import jax.numpy as jnp
from jax import lax
from jax.experimental import pallas as pl
from jax.experimental.pallas import tpu as pltpu
from jax.experimental.pallas import tpu_sc as plsc

N = 10000
E = 320000
D = 128

NC = 2
NS = 16
CHUNK = 128
K2 = 160
E_PAD = NS * K2 * CHUNK
NBUF = 1

G_ROWS = 10240
HALF = G_ROWS // NC
ACC_H = HALF + CHUNK
DUMMY = HALF
ACC_PER_TILE = ACC_H // NS
DEG_PER_TILE = G_ROWS // NS
TCB = 1024


def _zero_vmem_2d(ref, rows, cols):
  zero = jnp.zeros((16,), jnp.float32)

  @pl.loop(0, rows)
  def _(r):
    for k in range(cols // 16):
      ref[r, pl.ds(k * 16, 16)] = zero


def _sc_deg_body(dst_hbm, out_hbm, idx_d, ones_v, acc, sem):
  cid = lax.axis_index("c")
  sid = lax.axis_index("s")

  pltpu.sync_copy(dst_hbm.at[sid], idx_d)

  _zero_vmem_2d(ones_v, CHUNK, D)
  base = sid * DEG_PER_TILE
  for t in range(DEG_PER_TILE // CHUNK):
    pltpu.sync_copy(ones_v, acc.at[pl.ds(base + t * CHUNK, CHUNK)])
  one = jnp.ones((16,), jnp.float32)

  @pl.loop(0, CHUNK)
  def _(r):
    for k in range(D // 16):
      ones_v[r, pl.ds(k * 16, 16)] = one

  plsc.subcore_barrier()

  GRP = 8
  half = K2 // NC
  cbase = cid * half

  @pl.loop(0, half, step=GRP)
  def _(j0):
    for b in range(GRP):
      pltpu.async_copy(ones_v, acc.at[idx_d.at[cbase + j0 + b]], sem,
                       add=True)
    for b in range(GRP):
      pltpu.make_async_copy(ones_v, acc.at[idx_d.at[cbase + j0 + b]],
                            sem).wait()

  plsc.subcore_barrier()
  pltpu.sync_copy(acc.at[pl.ds(base, DEG_PER_TILE)],
                  out_hbm.at[cid, pl.ds(base, DEG_PER_TILE)])


def _sc_agg_body(g_hbm, src_hbm, dst_hbm, out_hbm, idx_s, idx_d, bufs, acc,
                 gs, ss):
  cid = lax.axis_index("c")
  sid = lax.axis_index("s")
  base_row = cid * HALF

  pltpu.sync_copy(src_hbm.at[sid], idx_s)
  pltpu.sync_copy(dst_hbm.at[sid], idx_d)

  @pl.loop(0, K2)
  def _(r):
    for k in range(CHUNK // 16):
      d = idx_d[r, pl.ds(k * 16, 16)]
      m = (d >= base_row) & (d < base_row + HALF)
      idx_d[r, pl.ds(k * 16, 16)] = jnp.where(m, d - base_row, DUMMY + (d & 127))

  _zero_vmem_2d(bufs.at[0], CHUNK, D)
  zb = sid * ACC_PER_TILE
  for t in range(ACC_PER_TILE // CHUNK):
    pltpu.sync_copy(bufs.at[0], acc.at[pl.ds(zb + t * CHUNK, CHUNK)])
  _REM = ACC_PER_TILE % CHUNK
  if _REM:
    pltpu.sync_copy(
        bufs.at[0, pl.ds(0, _REM)],
        acc.at[pl.ds(zb + (ACC_PER_TILE // CHUNK) * CHUNK, _REM)])
  plsc.subcore_barrier()

  joff0 = cid * (K2 // 2)
  for b in range(NBUF):
    pltpu.async_copy(g_hbm.at[idx_s.at[joff0 + b]], bufs.at[b], gs[b])

  joff = cid * (K2 // 2)

  def wrap(j):
    jw = j + joff
    return jnp.where(jw >= K2, jw - K2, jw)

  @pl.loop(0, K2, step=NBUF)
  def _(j0):
    for b in range(NBUF):
      j = j0 + b
      jw = wrap(j)
      pltpu.make_async_copy(g_hbm.at[idx_s.at[jw]], bufs.at[b], gs[b]).wait()
      pltpu.async_copy(bufs.at[b], acc.at[idx_d.at[jw]], ss[b], add=True)
      pltpu.make_async_copy(bufs.at[b], acc.at[idx_d.at[jw]], ss[b]).wait()
      jn = j + NBUF

      @pl.when(jn < K2)
      def _():
        pltpu.async_copy(g_hbm.at[idx_s.at[wrap(jn)]], bufs.at[b], gs[b])

  plsc.subcore_barrier()
  pltpu.sync_copy(acc.at[pl.ds(zb, ACC_PER_TILE)],
                  out_hbm.at[cid, pl.ds(zb, ACC_PER_TILE)])


def _make_sc_kernels():
  mesh = plsc.VectorSubcoreMesh(core_axis_name="c", subcore_axis_name="s")
  deg_k = functools.partial(
      pl.kernel,
      out_type=pltpu.HBM((NC, G_ROWS, D), jnp.float32),
      mesh=mesh,
      scratch_types=[
          pltpu.VMEM((K2, CHUNK), jnp.int32),
          pltpu.VMEM((CHUNK, D), jnp.float32),
          pltpu.VMEM_SHARED((G_ROWS, D), jnp.float32),
          pltpu.SemaphoreType.DMA,
      ],
  )(_sc_deg_body)

  agg_k = functools.partial(
      pl.kernel,
      out_type=pltpu.HBM((NC, ACC_H, D), jnp.float32),
      mesh=mesh,
      scratch_types=[
          pltpu.VMEM((K2, CHUNK), jnp.int32),
          pltpu.VMEM((K2, CHUNK), jnp.int32),
          pltpu.VMEM((NBUF, CHUNK, D), jnp.float32),
          pltpu.VMEM_SHARED((ACC_H, D), jnp.float32),
          [pltpu.SemaphoreType.DMA] * NBUF,
          [pltpu.SemaphoreType.DMA] * NBUF,
      ],
  )(_sc_agg_body)
  return deg_k, agg_k


_SC_DEG, _SC_AGG = _make_sc_kernels()


def _row_mask(i):
  rows = i * TCB + lax.broadcasted_iota(jnp.int32, (TCB, 1), 0)
  return rows < N


def _tc0_body(x_ref, w_ref, dga_ref, g_ref, dinv_ref):
  i = pl.program_id(0)
  deg = dga_ref[0, :, 0:1] + dga_ref[1, :, 0:1] + 1.0
  dinv = lax.rsqrt(deg)
  valid = _row_mask(i)
  h = jnp.dot(x_ref[...], w_ref[...], preferred_element_type=jnp.float32)
  g_ref[...] = jnp.where(valid, dinv * h, 0.0)
  dinv_ref[...] = jnp.where(valid, dinv, 0.0)


def _tc_mid_body(acc_ref, g_ref, dinv_ref, w_ref, b_ref, z_ref, gout_ref):
  i = pl.program_id(0)
  dinv = dinv_ref[...]
  z = dinv * (acc_ref[0] + g_ref[...]) + b_ref[...]
  z_ref[...] = z
  h = z * (1.0 / (1.0 + jnp.exp(-z)))
  g2 = dinv * jnp.dot(h, w_ref[...], preferred_element_type=jnp.float32)
  gout_ref[...] = jnp.where(_row_mask(i), g2, 0.0)


_GRID = G_ROWS // TCB
_BLK_PER_CORE = HALF // TCB

_full_w = pl.BlockSpec((D, D), lambda i: (0, 0))
_full_b = pl.BlockSpec((1, D), lambda i: (0, 0))
_rows_feat = pl.BlockSpec((TCB, D), lambda i: (i, 0))
_rows_one = pl.BlockSpec((TCB, 1), lambda i: (i, 0))
_rows_acc = pl.BlockSpec(
    (1, TCB, D), lambda i: (i // _BLK_PER_CORE, i % _BLK_PER_CORE, 0))
_rows_deg = pl.BlockSpec((NC, TCB, D), lambda i: (0, i, 0))

_TC0 = pl.pallas_call(
    _tc0_body,
    grid=(_GRID,),
    in_specs=[_rows_feat, _full_w, _rows_deg],
    out_specs=[_rows_feat, _rows_one],
    out_shape=[
        jax.ShapeDtypeStruct((G_ROWS, D), jnp.float32),
        jax.ShapeDtypeStruct((G_ROWS, 1), jnp.float32),
    ],
)

_TC_MID = pl.pallas_call(
    _tc_mid_body,
    grid=(_GRID,),
    in_specs=[_rows_acc, _rows_feat, _rows_one, _full_w, _full_b],
    out_specs=[_rows_feat, _rows_feat],
    out_shape=[
        jax.ShapeDtypeStruct((G_ROWS, D), jnp.float32),
        jax.ShapeDtypeStruct((G_ROWS, D), jnp.float32),
    ],
)


def kernel(x, edge_index, W0, b0, W1, b1, W2, b2, W3, b3, W4, b4):
  src = edge_index[0].astype(jnp.int32)
  dst = edge_index[1].astype(jnp.int32)
  pad = N + jnp.arange(E_PAD - E, dtype=jnp.int32) % (G_ROWS - N)
  src3 = jnp.concatenate([src, pad]).reshape(NS, K2, CHUNK)
  dst3 = jnp.concatenate([dst, pad]).reshape(NS, K2, CHUNK)

  degacc = _SC_DEG(dst3)
  g0, dinv = _TC0(x, W0, degacc)

  Ws = jnp.stack([W1, W2, W3, W4, W4])
  bs = jnp.stack([b0, b1, b2, b3, b4]).reshape(5, 1, D)

  def step(carry, wb):
    g, _ = carry
    W_next, b_i = wb
    acc = _SC_AGG(g, src3, dst3)
    z, g_new = _TC_MID(acc, g, dinv, W_next, b_i)
    return (g_new, z), None

  (_, z_last), _ = lax.scan(step, (g0, jnp.zeros_like(g0)), (Ws, bs))
  return z_last[:N]


# speedup vs baseline: 1.7868x; 1.7868x over previous
import functools

import jax
import jax.numpy as jnp
from jax import lax
from jax.experimental import pallas as pl
from jax.experimental.pallas import tpu as pltpu
from jax.experimental.pallas import tpu_sc as plsc

N = 10000
E = 320000
D = 128

NC = 2
NS = 16
CHUNK = 128
K2 = 160
E_PAD = NS * K2 * CHUNK
NBUF = 4
GSEG = 8
NSEG = K2 // GSEG

G_ROWS = 10240
HALF = G_ROWS // NC
ACC_H = HALF + CHUNK
DUMMY = HALF
ACC_PER_TILE = ACC_H // NS
DEG_PER_TILE = G_ROWS // NS
TCB = 1024


def _zero_vmem_2d(ref, rows, cols):
  zero = jnp.zeros((16,), jnp.float32)

  @pl.loop(0, rows)
  def _(r):
    for k in range(cols // 16):
      ref[r, pl.ds(k * 16, 16)] = zero


def _sc_deg_body(dst_hbm, out_hbm, idx_d, ones_v, acc, sem):
  cid = lax.axis_index("c")
  sid = lax.axis_index("s")

  pltpu.sync_copy(dst_hbm.at[sid], idx_d)

  _zero_vmem_2d(ones_v, CHUNK, D)
  base = sid * DEG_PER_TILE
  for t in range(DEG_PER_TILE // CHUNK):
    pltpu.sync_copy(ones_v, acc.at[pl.ds(base + t * CHUNK, CHUNK)])
  one = jnp.ones((16,), jnp.float32)

  @pl.loop(0, CHUNK)
  def _(r):
    for k in range(D // 16):
      ones_v[r, pl.ds(k * 16, 16)] = one

  plsc.subcore_barrier()

  GRP = 8
  half = K2 // NC
  cbase = cid * half

  @pl.loop(0, half, step=GRP)
  def _(j0):
    for b in range(GRP):
      pltpu.async_copy(ones_v, acc.at[idx_d.at[cbase + j0 + b]], sem,
                       add=True)
    for b in range(GRP):
      pltpu.make_async_copy(ones_v, acc.at[idx_d.at[cbase + j0 + b]],
                            sem).wait()

  plsc.subcore_barrier()
  pltpu.sync_copy(acc.at[pl.ds(base, DEG_PER_TILE)],
                  out_hbm.at[cid, pl.ds(base, DEG_PER_TILE)])


def _sc_agg_body(g_hbm, src_hbm, dst_hbm, out_hbm, idx_sg, idx_d, bufs, acc,
                 sgs, gs, ss):
  cid = lax.axis_index("c")
  sid = lax.axis_index("s")
  base_row = cid * HALF

  pltpu.sync_copy(dst_hbm.at[cid, sid], idx_d)

  @pl.loop(0, K2)
  def _(r):
    for k in range(CHUNK // 16):
      d = idx_d[r, pl.ds(k * 16, 16)]
      m = (d >= base_row) & (d < base_row + HALF)
      idx_d[r, pl.ds(k * 16, 16)] = jnp.where(m, d - base_row, DUMMY + (d & 127))

  _zero_vmem_2d(bufs.at[0], CHUNK, D)
  zb = sid * ACC_PER_TILE
  for t in range(ACC_PER_TILE // CHUNK):
    pltpu.sync_copy(bufs.at[0], acc.at[pl.ds(zb + t * CHUNK, CHUNK)])
  _REM = ACC_PER_TILE % CHUNK
  if _REM:
    pltpu.sync_copy(
        bufs.at[0, pl.ds(0, _REM)],
        acc.at[pl.ds(zb + (ACC_PER_TILE // CHUNK) * CHUNK, _REM)])
  plsc.subcore_barrier()

  pltpu.async_copy(src_hbm.at[cid, sid, pl.ds(0, GSEG)], idx_sg.at[0],
                   sgs[0])
  pltpu.async_copy(src_hbm.at[cid, sid, pl.ds(GSEG, GSEG)], idx_sg.at[1],
                   sgs[1])
  pltpu.make_async_copy(src_hbm.at[cid, sid, pl.ds(0, GSEG)], idx_sg.at[0],
                        sgs[0]).wait()
  for b in range(NBUF):
    pltpu.async_copy(g_hbm.at[idx_sg.at[0, b]], bufs.at[b], gs[b])

  def _segment(s, slot, nslot):
    for r in range(GSEG):
      b = r % NBUF
      j = s * GSEG + r
      pltpu.make_async_copy(g_hbm.at[idx_sg.at[slot, r]], bufs.at[b],
                            gs[b]).wait()
      pltpu.async_copy(bufs.at[b], acc.at[idx_d.at[j]], ss[b], add=True)
      pltpu.make_async_copy(bufs.at[b], acc.at[idx_d.at[j]], ss[b]).wait()
      if r == GSEG - NBUF:
        @pl.when(s < NSEG - 1)
        def _():
          pltpu.make_async_copy(
              src_hbm.at[cid, sid, pl.ds((s + 1) * GSEG, GSEG)],
              idx_sg.at[nslot], sgs[nslot]).wait()
      jn = j + NBUF

      if r < GSEG - NBUF:
        @pl.when(jn < K2)
        def _():
          pltpu.async_copy(g_hbm.at[idx_sg.at[slot, r + NBUF]], bufs.at[b],
                           gs[b])
      else:
        @pl.when(jn < K2)
        def _():
          pltpu.async_copy(
              g_hbm.at[idx_sg.at[nslot, r + NBUF - GSEG]], bufs.at[b], gs[b])

    @pl.when(s < NSEG - 2)
    def _():
      pltpu.async_copy(src_hbm.at[cid, sid, pl.ds((s + 2) * GSEG, GSEG)],
                       idx_sg.at[slot], sgs[slot])

  @pl.loop(0, NSEG, step=2)
  def _(s0):
    _segment(s0, 0, 1)
    _segment(s0 + 1, 1, 0)

  plsc.subcore_barrier()
  pltpu.sync_copy(acc.at[pl.ds(zb, ACC_PER_TILE)],
                  out_hbm.at[cid, pl.ds(zb, ACC_PER_TILE)])


def _make_sc_kernels():
  mesh = plsc.VectorSubcoreMesh(core_axis_name="c", subcore_axis_name="s")
  deg_k = functools.partial(
      pl.kernel,
      out_type=pltpu.HBM((NC, G_ROWS, D), jnp.float32),
      mesh=mesh,
      scratch_types=[
          pltpu.VMEM((K2, CHUNK), jnp.int32),
          pltpu.VMEM((CHUNK, D), jnp.float32),
          pltpu.VMEM_SHARED((G_ROWS, D), jnp.float32),
          pltpu.SemaphoreType.DMA,
      ],
  )(_sc_deg_body)

  agg_k = functools.partial(
      pl.kernel,
      out_type=pltpu.HBM((NC, ACC_H, D), jnp.float32),
      mesh=mesh,
      scratch_types=[
          pltpu.VMEM((2, GSEG, CHUNK), jnp.int32),
          pltpu.VMEM((K2, CHUNK), jnp.int32),
          pltpu.VMEM((NBUF, CHUNK, D), jnp.float32),
          pltpu.VMEM_SHARED((ACC_H, D), jnp.float32),
          [pltpu.SemaphoreType.DMA] * 2,
          [pltpu.SemaphoreType.DMA] * NBUF,
          [pltpu.SemaphoreType.DMA] * NBUF,
      ],
  )(_sc_agg_body)
  return deg_k, agg_k


_SC_DEG, _SC_AGG = _make_sc_kernels()


def _row_mask(i):
  rows = i * TCB + lax.broadcasted_iota(jnp.int32, (TCB, 1), 0)
  return rows < N


def _tc0_body(x_ref, w_ref, dga_ref, g_ref, dinv_ref):
  i = pl.program_id(0)
  deg = dga_ref[0, :, 0:1] + dga_ref[1, :, 0:1] + 1.0
  dinv = lax.rsqrt(deg)
  valid = _row_mask(i)
  h = jnp.dot(x_ref[...], w_ref[...], preferred_element_type=jnp.float32)
  g_ref[...] = jnp.where(valid, dinv * h, 0.0)
  dinv_ref[...] = jnp.where(valid, dinv, 0.0)


def _tc_mid_body(acc_ref, g_ref, dinv_ref, w_ref, b_ref, z_ref, gout_ref):
  i = pl.program_id(0)
  dinv = dinv_ref[...]
  z = dinv * (acc_ref[0] + g_ref[...]) + b_ref[...]
  z_ref[...] = z
  h = z * (1.0 / (1.0 + jnp.exp(-z)))
  g2 = dinv * jnp.dot(h, w_ref[...], preferred_element_type=jnp.float32)
  gout_ref[...] = jnp.where(_row_mask(i), g2, 0.0)


_GRID = G_ROWS // TCB
_BLK_PER_CORE = HALF // TCB

_full_w = pl.BlockSpec((D, D), lambda i: (0, 0))
_full_b = pl.BlockSpec((1, D), lambda i: (0, 0))
_rows_feat = pl.BlockSpec((TCB, D), lambda i: (i, 0))
_rows_one = pl.BlockSpec((TCB, 1), lambda i: (i, 0))
_rows_acc = pl.BlockSpec(
    (1, TCB, D), lambda i: (i // _BLK_PER_CORE, i % _BLK_PER_CORE, 0))
_rows_deg = pl.BlockSpec((NC, TCB, D), lambda i: (0, i, 0))

_TC0 = pl.pallas_call(
    _tc0_body,
    grid=(_GRID,),
    in_specs=[_rows_feat, _full_w, _rows_deg],
    out_specs=[_rows_feat, _rows_one],
    out_shape=[
        jax.ShapeDtypeStruct((G_ROWS, D), jnp.float32),
        jax.ShapeDtypeStruct((G_ROWS, 1), jnp.float32),
    ],
)

_TC_MID = pl.pallas_call(
    _tc_mid_body,
    grid=(_GRID,),
    in_specs=[_rows_acc, _rows_feat, _rows_one, _full_w, _full_b],
    out_specs=[_rows_feat, _rows_feat],
    out_shape=[
        jax.ShapeDtypeStruct((G_ROWS, D), jnp.float32),
        jax.ShapeDtypeStruct((G_ROWS, D), jnp.float32),
    ],
)


def kernel(x, edge_index, W0, b0, W1, b1, W2, b2, W3, b3, W4, b4):
  src = edge_index[0].astype(jnp.int32)
  dst = edge_index[1].astype(jnp.int32)
  pad = N + jnp.arange(E_PAD - E, dtype=jnp.int32) % (G_ROWS - N)
  src3 = jnp.concatenate([src, pad]).reshape(NS, K2, CHUNK)
  dst3 = jnp.concatenate([dst, pad]).reshape(NS, K2, CHUNK)

  src3w = jnp.stack([src3, jnp.roll(src3, -(K2 // 2), axis=1)])
  dst3w = jnp.stack([dst3, jnp.roll(dst3, -(K2 // 2), axis=1)])

  degacc = _SC_DEG(dst3)
  g0, dinv = _TC0(x, W0, degacc)

  Ws = jnp.stack([W1, W2, W3, W4, W4])
  bs = jnp.stack([b0, b1, b2, b3, b4]).reshape(5, 1, D)

  def step(carry, wb):
    g, _ = carry
    W_next, b_i = wb
    acc = _SC_AGG(g, src3w, dst3w)
    z, g_new = _TC_MID(acc, g, dinv, W_next, b_i)
    return (g_new, z), None

  (_, z_last), _ = lax.scan(step, (g0, jnp.zeros_like(g0)), (Ws, bs))
  return z_last[:N]
